# baseline (speedup 1.0000x reference)
import jax
import jax.numpy as jnp
from jax import lax
from jax.experimental import pallas as pl
from jax.experimental.pallas import tpu as pltpu
from jax.experimental.pallas import tpu_sc as plsc

_VOCAB = 1000000
_EMBED = 64
_BATCH = 16384
_K = 5

_NC = 2
_NS = 16
_NW = _NC * _NS
_BPW = _BATCH // _NW
_CHUNK = 128
_NCHUNK = _BPW // _CHUNK
_GRP = 16
_NGRP = _CHUNK // _GRP


def _skipgram_body(tidx_hbm, cidx_hbm, ttab_hbm, ctab_hbm, out_hbm,
                   tidx_v, cidx_v, trows, crows, out_v, sem):
  wid = lax.axis_index("s") * _NC + lax.axis_index("c")

  pltpu.sync_copy(tidx_hbm.at[pl.ds(wid * _BPW, _BPW)], tidx_v)
  pltpu.sync_copy(cidx_hbm.at[pl.ds(wid * _BPW * _K, _BPW * _K)], cidx_v)

  iota = lax.iota(jnp.int32, _GRP)

  for c in range(_NCHUNK):
    copies = [
        pltpu.async_copy(
            ttab_hbm.at[tidx_v.at[pl.ds(c * _CHUNK, _CHUNK)]],
            trows, sem)
    ]
    for r in range(_K):
      copies.append(
          pltpu.async_copy(
              ctab_hbm.at[cidx_v.at[pl.ds((c * _K + r) * _CHUNK, _CHUNK)]],
              crows.at[pl.ds(r * _CHUNK, _CHUNK)], sem))
    for cp in copies:
      cp.wait()

    for g in range(_NGRP):
      lane_b = g * _GRP + iota
      crow0 = lane_b * _K

      def body(e, accs):
        e_vec = jnp.full((_GRP,), e, jnp.int32)
        we = plsc.load_gather(trows, [lane_b, e_vec])
        return tuple(
            accs[j] + plsc.load_gather(crows, [crow0 + j, e_vec]) * we
            for j in range(_K))

      zero = jnp.zeros((_GRP,), jnp.float32)
      accs = lax.fori_loop(0, _EMBED, body, (zero,) * _K)

      obase = (c * _CHUNK + lane_b) * _K
      for j in range(_K):
        plsc.store_scatter(out_v, [obase + j], accs[j])

  pltpu.sync_copy(out_v, out_hbm.at[pl.ds(wid * _BPW * _K, _BPW * _K)])


@jax.jit
def _skipgram(tidx, cidx, ttab, ctab):
  mesh = plsc.VectorSubcoreMesh(core_axis_name="c", subcore_axis_name="s",
                                num_cores=_NC, num_subcores=_NS)
  kern = pl.kernel(
      _skipgram_body,
      out_type=jax.ShapeDtypeStruct((_BATCH * _K,), jnp.float32),
      mesh=mesh,
      compiler_params=pltpu.CompilerParams(needs_layout_passes=False,
                                           use_tc_tiling_on_sc=False),
      scratch_types=[
          pltpu.VMEM((_BPW,), jnp.int32),
          pltpu.VMEM((_BPW * _K,), jnp.int32),
          pltpu.VMEM((_CHUNK, _EMBED), jnp.float32),
          pltpu.VMEM((_CHUNK * _K, _EMBED), jnp.float32),
          pltpu.VMEM((_BPW * _K,), jnp.float32),
          pltpu.SemaphoreType.DMA,
      ],
  )
  return kern(tidx, cidx, ttab, ctab)


def kernel(target, context, target_table, context_table):
  tidx = target.reshape(_BATCH)
  cidx = context.reshape(_BATCH * _K)
  out = _skipgram(tidx, cidx, target_table, context_table)
  return out.reshape(_BATCH, _K)
